# Initial kernel scaffold; baseline (speedup 1.0000x reference)
import functools

import jax
import jax.numpy as jnp
from jax import lax
from jax.experimental import pallas as pl
from jax.experimental.pallas import tpu as pltpu
from jax.experimental.pallas import tpu_sc as plsc

EMBED_DIM = 32

IDX_ROW = 128
ROWS_PER_CHUNK = 8
CHUNK = IDX_ROW * ROWS_PER_CHUNK

NUM_CORES = 2
NUM_SUBCORES = 16
NW = NUM_CORES * NUM_SUBCORES


def _sc_gather(idx2d, table, n_rows):
    rows_per_w = n_rows // NW
    idx_rows_per_w = rows_per_w // IDX_ROW
    n_chunks = idx_rows_per_w // ROWS_PER_CHUNK

    mesh = plsc.VectorSubcoreMesh(core_axis_name="c", subcore_axis_name="s")

    @functools.partial(
        pl.kernel,
        mesh=mesh,
        out_type=jax.ShapeDtypeStruct((n_rows, EMBED_DIM), jnp.float32),
        scratch_types=[
            pltpu.VMEM((idx_rows_per_w, IDX_ROW), jnp.int32),
            pltpu.VMEM((CHUNK, EMBED_DIM), jnp.float32),
            pltpu.SemaphoreType.DMA,
        ],
    )
    def body(idx_hbm, table_hbm, out_hbm, idx_v, rows_v, sem):
        wid = lax.axis_index("s") * NUM_CORES + lax.axis_index("c")
        idx_row_base = wid * idx_rows_per_w
        out_base = wid * rows_per_w

        pltpu.sync_copy(idx_hbm.at[pl.ds(idx_row_base, idx_rows_per_w)], idx_v)

        def chunk_body(c, carry):
            copies = []
            for j in range(ROWS_PER_CHUNK):
                copies.append(
                    pltpu.async_copy(
                        table_hbm.at[idx_v.at[c * ROWS_PER_CHUNK + j]],
                        rows_v.at[pl.ds(j * IDX_ROW, IDX_ROW)],
                        sem,
                    )
                )
            for cp in copies:
                cp.wait()
            pltpu.sync_copy(
                rows_v, out_hbm.at[pl.ds(out_base + c * CHUNK, CHUNK)]
            )
            return carry

        lax.fori_loop(0, n_chunks, chunk_body, 0)

    return body(idx2d, table)


def kernel(x, table):
    batch, n_fields = x.shape
    n_rows = batch * n_fields
    idx2d = x.reshape(n_rows // IDX_ROW, IDX_ROW).astype(jnp.int32)
    out = _sc_gather(idx2d, table, n_rows)
    return out.reshape(batch, n_fields, EMBED_DIM)


# speedup vs baseline: 1.5590x; 1.5590x over previous
import functools

import jax
import jax.numpy as jnp
from jax import lax
from jax.experimental import pallas as pl
from jax.experimental.pallas import tpu as pltpu
from jax.experimental.pallas import tpu_sc as plsc

EMBED_DIM = 32

IDX_ROW = 128
ROWS_PER_CHUNK = 8
CHUNK = IDX_ROW * ROWS_PER_CHUNK

NUM_CORES = 2
NUM_SUBCORES = 16
NW = NUM_CORES * NUM_SUBCORES


def _sc_gather(idx2d, table, n_rows):
    rows_per_w = n_rows // NW
    idx_rows_per_w = rows_per_w // IDX_ROW
    n_chunks = idx_rows_per_w // ROWS_PER_CHUNK

    mesh = plsc.VectorSubcoreMesh(core_axis_name="c", subcore_axis_name="s")

    @functools.partial(
        pl.kernel,
        mesh=mesh,
        out_type=jax.ShapeDtypeStruct((n_rows, EMBED_DIM), jnp.float32),
        compiler_params=pltpu.CompilerParams(use_tc_tiling_on_sc=False),
        scratch_types=[
            pltpu.VMEM((idx_rows_per_w, IDX_ROW), jnp.int32),
            pltpu.VMEM((CHUNK, EMBED_DIM), jnp.float32),
            pltpu.SemaphoreType.DMA,
        ],
    )
    def body(idx_hbm, table_hbm, out_hbm, idx_v, rows_v, sem):
        wid = lax.axis_index("s") * NUM_CORES + lax.axis_index("c")
        idx_row_base = wid * idx_rows_per_w
        out_base = wid * rows_per_w

        pltpu.sync_copy(idx_hbm.at[pl.ds(idx_row_base, idx_rows_per_w)], idx_v)

        def chunk_body(c, carry):
            copies = []
            for j in range(ROWS_PER_CHUNK):
                copies.append(
                    pltpu.async_copy(
                        table_hbm.at[idx_v.at[c * ROWS_PER_CHUNK + j]],
                        rows_v.at[pl.ds(j * IDX_ROW, IDX_ROW)],
                        sem,
                    )
                )
            for cp in copies:
                cp.wait()
            pltpu.sync_copy(
                rows_v, out_hbm.at[pl.ds(out_base + c * CHUNK, CHUNK)]
            )
            return carry

        lax.fori_loop(0, n_chunks, chunk_body, 0)

    return body(idx2d, table)


def kernel(x, table):
    batch, n_fields = x.shape
    n_rows = batch * n_fields
    idx2d = x.reshape(n_rows // IDX_ROW, IDX_ROW).astype(jnp.int32)
    out = _sc_gather(idx2d, table, n_rows)
    return out.reshape(batch, n_fields, EMBED_DIM)


# speedup vs baseline: 1.5687x; 1.0062x over previous
import functools

import jax
import jax.numpy as jnp
from jax import lax
from jax.experimental import pallas as pl
from jax.experimental.pallas import tpu as pltpu
from jax.experimental.pallas import tpu_sc as plsc

EMBED_DIM = 32

NUM_CORES = 2
NUM_SUBCORES = 16
NW = NUM_CORES * NUM_SUBCORES

CHUNK = 1664


def _sc_gather(idx, table, n_rows):
    rows_per_w = n_rows // NW
    n_chunks = rows_per_w // CHUNK

    mesh = plsc.VectorSubcoreMesh(core_axis_name="c", subcore_axis_name="s")

    @functools.partial(
        pl.kernel,
        mesh=mesh,
        out_type=jax.ShapeDtypeStruct((n_rows, EMBED_DIM), jnp.float32),
        compiler_params=pltpu.CompilerParams(use_tc_tiling_on_sc=False),
        scratch_types=[
            pltpu.VMEM((rows_per_w,), jnp.int32),
            pltpu.VMEM((CHUNK, EMBED_DIM), jnp.float32),
            pltpu.SemaphoreType.DMA,
        ],
    )
    def body(idx_hbm, table_hbm, out_hbm, idx_v, rows_v, sem):
        wid = lax.axis_index("s") * NUM_CORES + lax.axis_index("c")
        base = wid * rows_per_w

        pltpu.sync_copy(idx_hbm.at[pl.ds(base, rows_per_w)], idx_v)

        def chunk_body(c, carry):
            pltpu.async_copy(
                table_hbm.at[idx_v.at[pl.ds(c * CHUNK, CHUNK)]],
                rows_v,
                sem,
            ).wait()
            pltpu.sync_copy(rows_v, out_hbm.at[pl.ds(base + c * CHUNK, CHUNK)])
            return carry

        lax.fori_loop(0, n_chunks, chunk_body, 0)

    return body(idx, table)


def kernel(x, table):
    batch, n_fields = x.shape
    n_rows = batch * n_fields
    idx = x.reshape(n_rows).astype(jnp.int32)
    out = _sc_gather(idx, table, n_rows)
    return out.reshape(batch, n_fields, EMBED_DIM)


# speedup vs baseline: 1.5766x; 1.0050x over previous
import functools

import jax
import jax.numpy as jnp
from jax import lax
from jax.experimental import pallas as pl
from jax.experimental.pallas import tpu as pltpu
from jax.experimental.pallas import tpu_sc as plsc

EMBED_DIM = 32

NUM_CORES = 2
NUM_SUBCORES = 16
NW = NUM_CORES * NUM_SUBCORES

CHUNK = 1664


def _sc_gather(idx, table, n_rows):
    rows_per_w = n_rows // NW
    n_chunks = rows_per_w // CHUNK

    mesh = plsc.VectorSubcoreMesh(core_axis_name="c", subcore_axis_name="s")

    @functools.partial(
        pl.kernel,
        mesh=mesh,
        out_type=jax.ShapeDtypeStruct((n_rows, EMBED_DIM), jnp.float32),
        compiler_params=pltpu.CompilerParams(use_tc_tiling_on_sc=False),
        scratch_types=[
            pltpu.VMEM((rows_per_w,), jnp.int32),
            pltpu.VMEM((CHUNK, EMBED_DIM), jnp.float32),
            pltpu.VMEM((CHUNK, EMBED_DIM), jnp.float32),
            pltpu.SemaphoreType.DMA,
            pltpu.SemaphoreType.DMA,
            pltpu.SemaphoreType.DMA,
            pltpu.SemaphoreType.DMA,
        ],
    )
    def body(idx_hbm, table_hbm, out_hbm, idx_v, rows0, rows1, g0, g1, o0, o1):
        wid = lax.axis_index("s") * NUM_CORES + lax.axis_index("c")
        base = wid * rows_per_w
        bufs = (rows0, rows1)
        gsem = (g0, g1)
        osem = (o0, o1)

        pltpu.sync_copy(idx_hbm.at[pl.ds(base, rows_per_w)], idx_v)

        def fire_gather(c):
            return pltpu.async_copy(
                table_hbm.at[idx_v.at[pl.ds(c * CHUNK, CHUNK)]],
                bufs[c % 2],
                gsem[c % 2],
            )

        gathers = [None] * n_chunks
        outs = [None] * n_chunks
        gathers[0] = fire_gather(0)
        for c in range(n_chunks):
            if c + 1 < n_chunks:
                if c >= 1:
                    outs[c - 1].wait()
                gathers[c + 1] = fire_gather(c + 1)
            gathers[c].wait()
            outs[c] = pltpu.async_copy(
                bufs[c % 2],
                out_hbm.at[pl.ds(base + c * CHUNK, CHUNK)],
                osem[c % 2],
            )
        outs[n_chunks - 2].wait()
        outs[n_chunks - 1].wait()

    return body(idx, table)


def kernel(x, table):
    batch, n_fields = x.shape
    n_rows = batch * n_fields
    idx = x.reshape(n_rows).astype(jnp.int32)
    out = _sc_gather(idx, table, n_rows)
    return out.reshape(batch, n_fields, EMBED_DIM)
